# baseline (speedup 1.0000x reference)
import functools

import jax
import jax.numpy as jnp
from jax import lax
from jax.experimental import pallas as pl
from jax.experimental.pallas import tpu as pltpu
from jax.experimental.pallas import tpu_sc as plsc

_NC = 2
_NS = 16
_NW = _NC * _NS
_L = 16

_CHUNK_ROWS = 4


def _make_sc_add(batch, seq, d):
    rows_per_w = seq // _NW
    n_chunks = rows_per_w // _CHUNK_ROWS
    chunk = _CHUNK_ROWS * d

    mesh = plsc.VectorSubcoreMesh(core_axis_name="c", subcore_axis_name="s")

    per_slot = 1 + 2 * batch
    vmem = [pltpu.VMEM((chunk,), jnp.float32)
            for _ in range(2 * per_slot)]
    sems = [pltpu.SemaphoreType.DMA for _ in range(4)]

    @functools.partial(
        pl.kernel,
        mesh=mesh,
        out_type=jax.ShapeDtypeStruct((batch, seq * d), jnp.float32),
        scratch_types=vmem + sems,
    )
    def sc_add(x_hbm, pos_hbm, out_hbm, *scratch):
        bufs, sem4 = scratch[: 2 * per_slot], scratch[2 * per_slot :]
        pos_v = (bufs[0], bufs[per_slot])
        x_v = (bufs[1 : 1 + batch], bufs[per_slot + 1 : per_slot + 1 + batch])
        o_v = (bufs[1 + batch : per_slot],
               bufs[per_slot + 1 + batch : 2 * per_slot])
        in_sem = (sem4[0], sem4[1])
        out_sem = (sem4[2], sem4[3])

        wid = lax.axis_index("s") * _NC + lax.axis_index("c")
        base = wid * rows_per_w * d

        def start_in(c, slot):
            off = base + c * chunk
            hs = [pltpu.async_copy(pos_hbm.at[pl.ds(off, chunk)],
                                   pos_v[slot], in_sem[slot])]
            for b in range(batch):
                hs.append(pltpu.async_copy(
                    x_hbm.at[b, pl.ds(off, chunk)],
                    x_v[slot][b], in_sem[slot]))
            return hs

        in_handles = [None, None]
        out_handles = [None, None]
        in_handles[0] = start_in(0, 0)
        for c in range(n_chunks):
            slot = c % 2
            nxt = 1 - slot
            if c + 1 < n_chunks:
                in_handles[nxt] = start_in(c + 1, nxt)
            for h in in_handles[slot]:
                h.wait()
            if out_handles[slot] is not None:
                for h in out_handles[slot]:
                    h.wait()
                out_handles[slot] = None

            @plsc.parallel_loop(0, chunk, step=_L, unroll=8)
            def _body(off, slot=slot):
                sl = pl.ds(off, _L)
                p = pos_v[slot][sl]
                for b in range(batch):
                    o_v[slot][b][sl] = x_v[slot][b][sl] + p

            off = base + c * chunk
            out_handles[slot] = [
                pltpu.async_copy(o_v[slot][b],
                                 out_hbm.at[b, pl.ds(off, chunk)],
                                 out_sem[slot])
                for b in range(batch)
            ]
        for hs in out_handles:
            if hs is not None:
                for h in hs:
                    h.wait()

    return sc_add


def kernel(x, pos_table):
    batch, seq, d = x.shape
    pos = pos_table[:seq]
    out = _make_sc_add(batch, seq, d)(
        x.reshape(batch, seq * d), pos.reshape(seq * d))
    return out.reshape(batch, seq, d)


# speedup vs baseline: 1.0360x; 1.0360x over previous
import functools

import jax
import jax.numpy as jnp
from jax import lax
from jax.experimental import pallas as pl
from jax.experimental.pallas import tpu as pltpu
from jax.experimental.pallas import tpu_sc as plsc

_NC = 2
_NS = 16
_NW = _NC * _NS
_L = 16

_CHUNK_ROWS = 8


def _make_sc_add(batch, seq, d):
    rows_per_w = seq // _NW
    n_chunks = rows_per_w // _CHUNK_ROWS
    chunk = _CHUNK_ROWS * d

    mesh = plsc.VectorSubcoreMesh(core_axis_name="c", subcore_axis_name="s")

    vmem = [
        pltpu.VMEM((chunk,), jnp.float32),
        pltpu.VMEM((batch, chunk), jnp.float32),
        pltpu.VMEM((chunk,), jnp.float32),
        pltpu.VMEM((batch, chunk), jnp.float32),
    ]
    sems = [pltpu.SemaphoreType.DMA for _ in range(4)]

    @functools.partial(
        pl.kernel,
        mesh=mesh,
        out_type=jax.ShapeDtypeStruct((batch, seq * d), jnp.float32),
        scratch_types=vmem + sems,
    )
    def sc_add(x_hbm, pos_hbm, out_hbm, *scratch):
        pos_v = (scratch[0], scratch[2])
        x_v = (scratch[1], scratch[3])
        sem4 = scratch[4:]
        in_sem = (sem4[0], sem4[1])
        out_sem = (sem4[2], sem4[3])

        wid = lax.axis_index("s") * _NC + lax.axis_index("c")
        base = wid * rows_per_w * d

        def start_in(c, slot):
            off = base + c * chunk
            return [
                pltpu.async_copy(pos_hbm.at[pl.ds(off, chunk)],
                                 pos_v[slot], in_sem[slot]),
                pltpu.async_copy(x_hbm.at[:, pl.ds(off, chunk)],
                                 x_v[slot], in_sem[slot]),
            ]

        in_handles = [None, None]
        out_handles = [None, None]
        in_handles[0] = start_in(0, 0)
        for c in range(n_chunks):
            slot = c % 2
            nxt = 1 - slot
            if c + 1 < n_chunks:
                if out_handles[nxt] is not None:
                    out_handles[nxt].wait()
                    out_handles[nxt] = None
                in_handles[nxt] = start_in(c + 1, nxt)
            for h in in_handles[slot]:
                h.wait()

            @plsc.parallel_loop(0, chunk, step=_L, unroll=8)
            def _body(off, slot=slot):
                sl = pl.ds(off, _L)
                p = pos_v[slot][sl]
                for b in range(batch):
                    x_v[slot][b, sl] = x_v[slot][b, sl] + p

            off = base + c * chunk
            out_handles[slot] = pltpu.async_copy(
                x_v[slot], out_hbm.at[:, pl.ds(off, chunk)], out_sem[slot])
        for h in out_handles:
            if h is not None:
                h.wait()

    return sc_add


def kernel(x, pos_table):
    batch, seq, d = x.shape
    pos = pos_table[:seq]
    out = _make_sc_add(batch, seq, d)(
        x.reshape(batch, seq * d), pos.reshape(seq * d))
    return out.reshape(batch, seq, d)
